# baseline (device time: 84326 ns/iter reference)
import jax
import jax.numpy as jnp
from jax import lax
from jax.experimental import pallas as pl
from jax.experimental.pallas import tpu as pltpu

N_DEV = 8
M = 4096
K = 4096
N = 8192
N_HALF = N // 2
BLK = M // N_DEV

SCHED = [[0], [1], [2, 3], [4, 5], [6], [7]]
CHUNKS = [(t, h) for grp in SCHED for h in range(2) for t in grp]
SENDS_AFTER_GROUP = {-1: [1, 2, 3], 0: [4], 1: [5], 2: [6, 7]}


def kernel(x, w_mat):
    def body(x_ref, w_ref, out_ref, xs_ref, recv_buf, wbuf, send_sems,
             recv_sems, w_sems):
        me = lax.axis_index("i")

        bsem = pltpu.get_barrier_semaphore()
        for i in range(1, N_DEV):
            peer = lax.rem(me + i, N_DEV)
            pl.semaphore_signal(
                bsem, inc=1, device_id=(peer,),
                device_id_type=pl.DeviceIdType.MESH,
            )
        pl.semaphore_wait(bsem, N_DEV - 1)

        xs_ref[...] = x_ref[...].astype(jnp.bfloat16)

        def start_send(i):
            d = lax.rem(me - i + N_DEV, N_DEV)
            rdma = pltpu.make_async_remote_copy(
                src_ref=xs_ref.at[pl.ds(d * BLK, BLK), :],
                dst_ref=recv_buf.at[i],
                send_sem=send_sems.at[i - 1],
                recv_sem=recv_sems.at[i],
                device_id=(d,),
                device_id_type=pl.DeviceIdType.MESH,
            )
            rdma.start()
            return rdma

        sends = [start_send(i) for i in SENDS_AFTER_GROUP[-1]]

        def w_copy(s):
            t, h = CHUNKS[s]
            j = lax.rem(me + t, N_DEV)
            return pltpu.make_async_copy(
                w_ref.at[pl.ds(j * BLK, BLK), pl.ds(h * N_HALF, N_HALF)],
                wbuf.at[s % 3],
                w_sems.at[s % 3],
            )

        def wait_recv(t):
            pltpu.make_async_remote_copy(
                src_ref=recv_buf.at[t],
                dst_ref=recv_buf.at[t],
                send_sem=send_sems.at[0],
                recv_sem=recv_sems.at[t],
                device_id=(me,),
                device_id_type=pl.DeviceIdType.MESH,
            ).wait_recv()

        for s in range(3):
            w_copy(s).start()
        started, consumed = 3, 0

        for g, grp in enumerate(SCHED):
            xblks = []
            for t in grp:
                if t == 0:
                    xblks.append(x_ref[pl.ds(me * BLK, BLK), :])
                else:
                    wait_recv(t)
                    xblks.append(recv_buf[t].astype(jnp.float32))
            for h in range(2):
                slots = list(range(consumed, consumed + len(grp)))
                for s in slots:
                    w_copy(s).wait()
                part = jnp.dot(xblks[0], wbuf[slots[0] % 3],
                               preferred_element_type=jnp.float32)
                for xb, s in zip(xblks[1:], slots[1:]):
                    part = part + jnp.dot(xb, wbuf[s % 3],
                                          preferred_element_type=jnp.float32)
                nsl = slice(h * N_HALF, (h + 1) * N_HALF)
                if g == 0:
                    out_ref[:, nsl] = part
                elif g == len(SCHED) - 1:
                    out_ref[:, nsl] = jnp.maximum(out_ref[:, nsl] + part, 0.0)
                else:
                    out_ref[:, nsl] = out_ref[:, nsl] + part
                consumed += len(grp)
                while started < len(CHUNKS) and started < consumed + 3:
                    w_copy(started).start()
                    started += 1
            for i in SENDS_AFTER_GROUP.get(g, []):
                sends.append(start_send(i))

        for rdma in sends:
            rdma.wait_send()

    return pl.pallas_call(
        body,
        out_shape=jax.ShapeDtypeStruct((BLK, N), jnp.float32),
        in_specs=[
            pl.BlockSpec(memory_space=pltpu.VMEM),
            pl.BlockSpec(memory_space=pl.ANY),
        ],
        out_specs=pl.BlockSpec(memory_space=pltpu.VMEM),
        scratch_shapes=[
            pltpu.VMEM((K, BLK), jnp.bfloat16),
            pltpu.VMEM((N_DEV, BLK, BLK), jnp.bfloat16),
            pltpu.VMEM((3, BLK, N_HALF), jnp.float32),
            pltpu.SemaphoreType.DMA((N_DEV - 1,)),
            pltpu.SemaphoreType.DMA((N_DEV,)),
            pltpu.SemaphoreType.DMA((3,)),
        ],
        compiler_params=pltpu.CompilerParams(
            collective_id=0,
            vmem_limit_bytes=63 * 1024 * 1024,
        ),
    )(x, w_mat)


# device time: 81501 ns/iter; 1.0347x vs baseline; 1.0347x over previous
import jax
import jax.numpy as jnp
from jax import lax
from jax.experimental import pallas as pl
from jax.experimental.pallas import tpu as pltpu

N_DEV = 8
M = 4096
K = 4096
N = 8192
N_HALF = N // 2
BLK = M // N_DEV

SCHED = [[t] for t in range(N_DEV)]
N_SPLIT = 4
N_CHUNK = N // N_SPLIT
W_BUFS = 6
CHUNKS = [(t, q) for grp in SCHED for q in range(N_SPLIT) for t in grp]
SENDS_AFTER_GROUP = {-1: [1, 2, 3], 0: [4], 1: [5], 2: [6], 3: [7]}


def kernel(x, w_mat):
    def body(x_ref, w_ref, out_ref, xs_ref, recv_buf, wbuf, send_sems,
             recv_sems, w_sems):
        me = lax.axis_index("i")

        bsem = pltpu.get_barrier_semaphore()
        for i in range(1, N_DEV):
            peer = lax.rem(me + i, N_DEV)
            pl.semaphore_signal(
                bsem, inc=1, device_id=(peer,),
                device_id_type=pl.DeviceIdType.MESH,
            )
        pl.semaphore_wait(bsem, N_DEV - 1)

        xs_ref[...] = x_ref[...].astype(jnp.bfloat16)

        def start_send(i):
            d = lax.rem(me - i + N_DEV, N_DEV)
            rdma = pltpu.make_async_remote_copy(
                src_ref=xs_ref.at[pl.ds(d * BLK, BLK), :],
                dst_ref=recv_buf.at[i],
                send_sem=send_sems.at[i - 1],
                recv_sem=recv_sems.at[i],
                device_id=(d,),
                device_id_type=pl.DeviceIdType.MESH,
            )
            rdma.start()
            return rdma

        sends = [start_send(i) for i in SENDS_AFTER_GROUP[-1]]

        def w_copy(s):
            t, h = CHUNKS[s]
            j = lax.rem(me + t, N_DEV)
            return pltpu.make_async_copy(
                w_ref.at[pl.ds(j * BLK, BLK), pl.ds(h * N_CHUNK, N_CHUNK)],
                wbuf.at[s % W_BUFS],
                w_sems.at[s % W_BUFS],
            )

        def wait_recv(t):
            pltpu.make_async_remote_copy(
                src_ref=recv_buf.at[t],
                dst_ref=recv_buf.at[t],
                send_sem=send_sems.at[0],
                recv_sem=recv_sems.at[t],
                device_id=(me,),
                device_id_type=pl.DeviceIdType.MESH,
            ).wait_recv()

        for s in range(W_BUFS):
            w_copy(s).start()
        started, consumed = W_BUFS, 0

        for g, grp in enumerate(SCHED):
            xblks = []
            for t in grp:
                if t == 0:
                    xblks.append(x_ref[pl.ds(me * BLK, BLK), :])
                else:
                    wait_recv(t)
                    xblks.append(recv_buf[t].astype(jnp.float32))
            for h in range(N_SPLIT):
                slots = list(range(consumed, consumed + len(grp)))
                for s in slots:
                    w_copy(s).wait()
                part = jnp.dot(xblks[0], wbuf[slots[0] % W_BUFS],
                               preferred_element_type=jnp.float32)
                for xb, s in zip(xblks[1:], slots[1:]):
                    part = part + jnp.dot(xb, wbuf[s % W_BUFS],
                                          preferred_element_type=jnp.float32)
                nsl = slice(h * N_CHUNK, (h + 1) * N_CHUNK)
                if g == 0:
                    out_ref[:, nsl] = part
                elif g == len(SCHED) - 1:
                    out_ref[:, nsl] = jnp.maximum(out_ref[:, nsl] + part, 0.0)
                else:
                    out_ref[:, nsl] = out_ref[:, nsl] + part
                consumed += len(grp)
                while started < len(CHUNKS) and started < consumed + W_BUFS:
                    w_copy(started).start()
                    started += 1
            for i in SENDS_AFTER_GROUP.get(g, []):
                sends.append(start_send(i))

        for rdma in sends:
            rdma.wait_send()

    return pl.pallas_call(
        body,
        out_shape=jax.ShapeDtypeStruct((BLK, N), jnp.float32),
        in_specs=[
            pl.BlockSpec(memory_space=pltpu.VMEM),
            pl.BlockSpec(memory_space=pl.ANY),
        ],
        out_specs=pl.BlockSpec(memory_space=pltpu.VMEM),
        scratch_shapes=[
            pltpu.VMEM((K, BLK), jnp.bfloat16),
            pltpu.VMEM((N_DEV, BLK, BLK), jnp.bfloat16),
            pltpu.VMEM((W_BUFS, BLK, N_CHUNK), jnp.float32),
            pltpu.SemaphoreType.DMA((N_DEV - 1,)),
            pltpu.SemaphoreType.DMA((N_DEV,)),
            pltpu.SemaphoreType.DMA((W_BUFS,)),
        ],
        compiler_params=pltpu.CompilerParams(
            collective_id=0,
            vmem_limit_bytes=63 * 1024 * 1024,
        ),
    )(x, w_mat)


# device time: 74289 ns/iter; 1.1351x vs baseline; 1.0971x over previous
import jax
import jax.numpy as jnp
from jax import lax
from jax.experimental import pallas as pl
from jax.experimental.pallas import tpu as pltpu

N_DEV = 8
M = 4096
K = 4096
N = 8192
N_HALF = N // 2
BLK = M // N_DEV

SCHED = [[t] for t in range(N_DEV)]
N_SPLIT = 2
N_CHUNK = N // N_SPLIT
W_BUFS = 3
CHUNKS = [(t, q) for grp in SCHED for q in range(N_SPLIT) for t in grp]
SENDS_AFTER_GROUP = {-1: [1, 2, 3], 0: [4], 1: [5], 2: [6], 3: [7]}


def kernel(x, w_mat):
    def body(x_ref, w_ref, out_ref, xs_ref, recv_buf, wbuf, send_sems,
             recv_sems, w_sems):
        me = lax.axis_index("i")

        bsem = pltpu.get_barrier_semaphore()
        for i in range(1, N_DEV):
            peer = lax.rem(me + i, N_DEV)
            pl.semaphore_signal(
                bsem, inc=1, device_id=(peer,),
                device_id_type=pl.DeviceIdType.MESH,
            )
        pl.semaphore_wait(bsem, N_DEV - 1)

        xs_ref[...] = x_ref[...].astype(jnp.bfloat16)

        def start_send(i):
            d = lax.rem(me - i + N_DEV, N_DEV)
            rdma = pltpu.make_async_remote_copy(
                src_ref=xs_ref.at[pl.ds(d * BLK, BLK), :],
                dst_ref=recv_buf.at[i],
                send_sem=send_sems.at[i - 1],
                recv_sem=recv_sems.at[i],
                device_id=(d,),
                device_id_type=pl.DeviceIdType.MESH,
            )
            rdma.start()
            return rdma

        sends = [start_send(i) for i in SENDS_AFTER_GROUP[-1]]

        def w_copy(s):
            t, h = CHUNKS[s]
            j = lax.rem(me + t, N_DEV)
            return pltpu.make_async_copy(
                w_ref.at[pl.ds(j * BLK, BLK), pl.ds(h * N_CHUNK, N_CHUNK)],
                wbuf.at[s % W_BUFS],
                w_sems.at[s % W_BUFS],
            )

        def wait_recv(t):
            pltpu.make_async_remote_copy(
                src_ref=recv_buf.at[t],
                dst_ref=recv_buf.at[t],
                send_sem=send_sems.at[0],
                recv_sem=recv_sems.at[t],
                device_id=(me,),
                device_id_type=pl.DeviceIdType.MESH,
            ).wait_recv()

        for s in range(W_BUFS):
            w_copy(s).start()
        started, consumed = W_BUFS, 0

        for g, grp in enumerate(SCHED):
            xblks = []
            for t in grp:
                if t == 0:
                    xblks.append(x_ref[pl.ds(me * BLK, BLK), :])
                else:
                    wait_recv(t)
                    xblks.append(recv_buf[t].astype(jnp.float32))
            for h in range(N_SPLIT):
                slots = list(range(consumed, consumed + len(grp)))
                for s in slots:
                    w_copy(s).wait()
                consumed += len(grp)
                part = jnp.dot(xblks[0], wbuf[slots[0] % W_BUFS],
                               preferred_element_type=jnp.float32)
                for xb, s in zip(xblks[1:], slots[1:]):
                    part = part + jnp.dot(xb, wbuf[s % W_BUFS],
                                          preferred_element_type=jnp.float32)
                nsl = slice(h * N_CHUNK, (h + 1) * N_CHUNK)
                if g == 0:
                    out_ref[:, nsl] = part
                elif g == len(SCHED) - 1:
                    out_ref[:, nsl] = jnp.maximum(out_ref[:, nsl] + part, 0.0)
                else:
                    out_ref[:, nsl] = out_ref[:, nsl] + part
                while started < len(CHUNKS) and started < consumed + W_BUFS:
                    w_copy(started).start()
                    started += 1
            for i in SENDS_AFTER_GROUP.get(g, []):
                sends.append(start_send(i))

        for rdma in sends:
            rdma.wait_send()

    return pl.pallas_call(
        body,
        out_shape=jax.ShapeDtypeStruct((BLK, N), jnp.float32),
        in_specs=[
            pl.BlockSpec(memory_space=pltpu.VMEM),
            pl.BlockSpec(memory_space=pl.ANY),
        ],
        out_specs=pl.BlockSpec(memory_space=pltpu.VMEM),
        scratch_shapes=[
            pltpu.VMEM((K, BLK), jnp.bfloat16),
            pltpu.VMEM((N_DEV, BLK, BLK), jnp.bfloat16),
            pltpu.VMEM((W_BUFS, BLK, N_CHUNK), jnp.float32),
            pltpu.SemaphoreType.DMA((N_DEV - 1,)),
            pltpu.SemaphoreType.DMA((N_DEV,)),
            pltpu.SemaphoreType.DMA((W_BUFS,)),
        ],
        compiler_params=pltpu.CompilerParams(
            collective_id=0,
            vmem_limit_bytes=63 * 1024 * 1024,
        ),
    )(x, w_mat)
